# baseline (speedup 1.0000x reference)
import dataclasses
import functools

import jax
import jax.numpy as jnp
from jax import lax
from jax.experimental import pallas as pl
from jax.experimental.pallas import tpu as pltpu
from jax.experimental.pallas import tpu_sc as plsc

NC = 2
NS = 16
NW = NC * NS
L = 16
F = 16
CHUNK = 128


def _mf_kernel_body(task_hbm, worker_hbm, tf_hbm, wf_hbm, out_hbm,
                    tidx_v, widx_v, trows_v, wrows_v, out_v, sems):
    b_per_w = tidx_v.shape[0]
    n_chunks = b_per_w // CHUNK
    wid = lax.axis_index("s") * NC + lax.axis_index("c")
    base = wid * b_per_w

    pltpu.sync_copy(task_hbm.at[pl.ds(base, b_per_w)], tidx_v)
    pltpu.sync_copy(worker_hbm.at[pl.ds(base, b_per_w)], widx_v)

    copies = []
    for c in range(n_chunks):
        sl = pl.ds(c * CHUNK, CHUNK)
        copies.append(pltpu.async_copy(
            tf_hbm.at[tidx_v.at[sl]], trows_v.at[sl], sems.at[2 * c]))
        copies.append(pltpu.async_copy(
            wf_hbm.at[widx_v.at[sl]], wrows_v.at[sl], sems.at[2 * c + 1]))
    for cp in copies:
        cp.wait()

    row_iota = lax.iota(jnp.int32, L)

    @pl.loop(0, b_per_w, step=L)
    def _(g):
        ridx = row_iota + g
        acc = jnp.zeros((L,), jnp.float32)
        for f in range(F):
            fidx = jnp.full((L,), f, jnp.int32)
            tcol = plsc.load_gather(trows_v, [ridx, fidx])
            wcol = plsc.load_gather(wrows_v, [ridx, fidx])
            acc = acc + tcol * wcol
        out_v[pl.ds(g, L)] = 1.0 / (1.0 + jnp.exp(-acc))

    pltpu.sync_copy(out_v, out_hbm.at[pl.ds(base, b_per_w)])


@jax.jit
def _mf_forward(task, worker, task_factors, worker_factors):
    B = task.shape[0]
    b_per_w = B // NW
    n_chunks = b_per_w // CHUNK
    mesh = plsc.VectorSubcoreMesh(core_axis_name="c", subcore_axis_name="s")
    cp = pltpu.CompilerParams(needs_layout_passes=False,
                              use_tc_tiling_on_sc=False)
    kern = functools.partial(
        pl.kernel,
        compiler_params=cp,
        out_type=jax.ShapeDtypeStruct((B,), jnp.float32),
        mesh=mesh,
        scratch_types=[
            pltpu.VMEM((b_per_w,), jnp.int32),
            pltpu.VMEM((b_per_w,), jnp.int32),
            pltpu.VMEM((b_per_w, F), jnp.float32),
            pltpu.VMEM((b_per_w, F), jnp.float32),
            pltpu.VMEM((b_per_w,), jnp.float32),
            pltpu.SemaphoreType.DMA((2 * n_chunks,)),
        ],
    )(_mf_kernel_body)
    return kern(task, worker, task_factors, worker_factors)


def kernel(task, worker, task_factors, worker_factors):
    return _mf_forward(task, worker, task_factors, worker_factors)


# speedup vs baseline: 3.6286x; 3.6286x over previous
import functools

import jax
import jax.numpy as jnp
from jax import lax
from jax.experimental import pallas as pl
from jax.experimental.pallas import tpu as pltpu
from jax.experimental.pallas import tpu_sc as plsc

NC = 2
NS = 16
NW = NC * NS
L = 16
F = 16
WCHUNK = 128


def _mf_kernel_body(task_hbm, worker_hbm, tfT_hbm, wfp_hbm, out_hbm,
                    tidx_v, widx_v, wblk_v, tring_v, wbuf_v,
                    trows_v, wrows_v, out_v, sems):
    b_per_w = tidx_v.shape[0]
    n_groups = b_per_w // L
    n_wchunks = b_per_w // WCHUNK
    n_cols = tfT_hbm.shape[1]
    wid = lax.axis_index("s") * NC + lax.axis_index("c")
    base = wid * b_per_w

    pltpu.sync_copy(task_hbm.at[pl.ds(base, b_per_w)], tidx_v)
    pltpu.sync_copy(worker_hbm.at[pl.ds(base, b_per_w)], widx_v)

    row_iota = lax.iota(jnp.int32, L)

    @pl.loop(0, b_per_w, step=L)
    def _(g):
        wblk_v[pl.ds(g, L)] = lax.shift_right_logical(widx_v[pl.ds(g, L)], 3)

    def w_start(c, buf):
        sl = pl.ds(c * WCHUNK, WCHUNK)
        return pltpu.async_copy(wfp_hbm.at[wblk_v.at[sl]], wbuf_v.at[buf],
                                sems.at[2 + buf])

    def w_extract(c, buf):
        @pl.loop(0, WCHUNK, step=L)
        def _(g):
            wv = widx_v[pl.ds(c * WCHUNK + g, L)]
            for j in range(L):
                sub = lax.bitwise_and(wv[j], 7)
                cidx = row_iota + sub * F
                ridx = jnp.full((L,), g + j, jnp.int32)
                wrow = plsc.load_gather(wbuf_v.at[buf], [ridx, cidx])
                wrows_v[pl.ds((c * WCHUNK + g + j) * F, F)] = wrow

    wcp = w_start(0, 0)
    for c in range(n_wchunks):
        nxt = w_start(c + 1, 1 - c % 2) if c + 1 < n_wchunks else None
        wcp.wait()
        w_extract(c, c % 2)
        wcp = nxt

    def t_fire(g, ring):
        tv = tidx_v[pl.ds(g * L, L)]
        for j in range(L):
            blk = pl.multiple_of(
                lax.shift_right_logical(tv[j], 7) * 128, 128)
            pltpu.async_copy(tfT_hbm.at[:, pl.ds(blk, 128)],
                             tring_v.at[ring, j], sems.at[ring])

    def t_drain(ring):
        for j in range(L):
            pltpu.make_async_copy(tfT_hbm.at[:, pl.ds(0, 128)],
                                  tring_v.at[ring, j], sems.at[ring]).wait()

    def t_extract(g, ring):
        tv = tidx_v[pl.ds(g * L, L)]
        for j in range(L):
            col = lax.bitwise_and(tv[j], 127)
            cidx = jnp.full((L,), col, jnp.int32)
            tcol = plsc.load_gather(tring_v.at[ring, j], [row_iota, cidx])
            trows_v[pl.ds((g * L + j) * F, F)] = tcol

    def t_body(g, ring):
        t_drain(ring)
        t_extract(g, ring)
        @pl.when(g + 2 < n_groups)
        def _():
            t_fire(g + 2, ring)

    t_fire(0, 0)
    t_fire(1, 1)

    @pl.loop(0, n_groups, step=2)
    def _(g):
        t_body(g, 0)
        t_body(g + 1, 1)

    @pl.loop(0, b_per_w, step=L)
    def _(p0):
        opos = row_iota + p0
        acc = jnp.zeros((L,), jnp.float32)
        for f in range(F):
            idx = opos * F + f
            tcol = plsc.load_gather(trows_v, [idx])
            wcol = plsc.load_gather(wrows_v, [idx])
            acc = acc + tcol * wcol
        out_v[pl.ds(p0, L)] = 1.0 / (1.0 + jnp.exp(-acc))

    pltpu.sync_copy(out_v, out_hbm.at[pl.ds(base, b_per_w)])


@jax.jit
def _mf_forward(task, worker, task_factors, worker_factors):
    B = task.shape[0]
    b_per_w = B // NW
    tfT = task_factors.T
    wfp = worker_factors.reshape(worker_factors.shape[0] // 8, 8 * F)
    mesh = plsc.VectorSubcoreMesh(core_axis_name="c", subcore_axis_name="s")
    cp = pltpu.CompilerParams(needs_layout_passes=False,
                              use_tc_tiling_on_sc=True)
    kern = functools.partial(
        pl.kernel,
        compiler_params=cp,
        out_type=jax.ShapeDtypeStruct((B,), jnp.float32),
        mesh=mesh,
        scratch_types=[
            pltpu.VMEM((b_per_w,), jnp.int32),
            pltpu.VMEM((b_per_w,), jnp.int32),
            pltpu.VMEM((b_per_w,), jnp.int32),
            pltpu.VMEM((2, L, F, 128), jnp.float32),
            pltpu.VMEM((2, WCHUNK, 8 * F), jnp.float32),
            pltpu.VMEM((b_per_w * F,), jnp.float32),
            pltpu.VMEM((b_per_w * F,), jnp.float32),
            pltpu.VMEM((b_per_w,), jnp.float32),
            pltpu.SemaphoreType.DMA((4,)),
        ],
    )(_mf_kernel_body)
    return kern(task, worker, tfT, wfp)


def kernel(task, worker, task_factors, worker_factors):
    return _mf_forward(task, worker, task_factors, worker_factors)


# speedup vs baseline: 3.9532x; 1.0895x over previous
import functools

import jax
import jax.numpy as jnp
from jax import lax
from jax.experimental import pallas as pl
from jax.experimental.pallas import tpu as pltpu
from jax.experimental.pallas import tpu_sc as plsc

NC = 2
NS = 16
NW = NC * NS
L = 16
F = 16
WCHUNK = 64
NRING = 3


def _mf_kernel_body(task_hbm, worker_hbm, tfT_hbm, wfp_hbm, out_hbm,
                    tidx_v, widx_v, wblk_v, tring_v, wbuf_v,
                    tg_v, wrows_v, out_v, sems):
    b_per_w = tidx_v.shape[0]
    n_groups = b_per_w // L
    n_wchunks = b_per_w // WCHUNK
    wid = lax.axis_index("s") * NC + lax.axis_index("c")
    base = wid * b_per_w

    pltpu.sync_copy(task_hbm.at[pl.ds(base, b_per_w)], tidx_v)
    pltpu.sync_copy(worker_hbm.at[pl.ds(base, b_per_w)], widx_v)

    row_iota = lax.iota(jnp.int32, L)
    lane16 = row_iota * F

    def t_fire(g, ring):
        tv = tidx_v[pl.ds(g * L, L)]
        for j in range(L):
            blk = pl.multiple_of(
                lax.shift_right_logical(tv[j], 7) * 128, 128)
            pltpu.async_copy(tfT_hbm.at[:, pl.ds(blk, 128)],
                             tring_v.at[ring, j], sems.at[ring])

    def t_drain(ring):
        for j in range(L):
            pltpu.make_async_copy(tfT_hbm.at[:, pl.ds(0, 128)],
                                  tring_v.at[ring, j], sems.at[ring]).wait()

    @pl.loop(0, b_per_w, step=L)
    def _(g):
        wblk_v[pl.ds(g, L)] = lax.shift_right_logical(widx_v[pl.ds(g, L)], 3)

    def w_start(c, buf):
        sl = pl.ds(c * WCHUNK, WCHUNK)
        return pltpu.async_copy(wfp_hbm.at[wblk_v.at[sl]], wbuf_v.at[buf],
                                sems.at[NRING + buf])

    def w_extract(c, buf):
        @pl.loop(0, WCHUNK, step=L)
        def _(g):
            wv = widx_v[pl.ds(c * WCHUNK + g, L)]
            for j in range(L):
                sub = lax.bitwise_and(wv[j], 7)
                cidx = row_iota + sub * F
                ridx = jnp.full((L,), g + j, jnp.int32)
                wrow = plsc.load_gather(wbuf_v.at[buf], [ridx, cidx])
                wrows_v[pl.ds((c * WCHUNK + g + j) * F, F)] = wrow

    wcp = w_start(0, 0)
    for r in range(NRING):
        t_fire(r, r)

    for c in range(n_wchunks):
        nxt = w_start(c + 1, 1 - c % 2) if c + 1 < n_wchunks else None
        wcp.wait()
        w_extract(c, c % 2)
        wcp = nxt

    def t_group(g, ring):
        t_drain(ring)
        tv = tidx_v[pl.ds(g * L, L)]
        for j in range(L):
            col = lax.bitwise_and(tv[j], 127)
            cidx = jnp.full((L,), col, jnp.int32)
            tcol = plsc.load_gather(tring_v.at[ring, j], [row_iota, cidx])
            tg_v[pl.ds(j * F, F)] = tcol

        @pl.when(g + NRING < n_groups)
        def _():
            t_fire(g + NRING, ring)

        acc = jnp.zeros((L,), jnp.float32)
        for f in range(F):
            tcol = plsc.load_gather(tg_v, [lane16 + f])
            wcol = plsc.load_gather(wrows_v, [lane16 + (g * L * F + f)])
            acc = acc + tcol * wcol
        out_v[pl.ds(g * L, L)] = 1.0 / (1.0 + jnp.exp(-acc))

    @pl.loop(0, n_groups, step=NRING)
    def _(g):
        for r in range(NRING):
            @pl.when(g + r < n_groups)
            def _(r=r):
                t_group(g + r, r)

    pltpu.sync_copy(out_v, out_hbm.at[pl.ds(base, b_per_w)])


@jax.jit
def _mf_forward(task, worker, task_factors, worker_factors):
    B = task.shape[0]
    b_per_w = B // NW
    tfT = task_factors.T
    wfp = worker_factors.reshape(worker_factors.shape[0] // 8, 8 * F)
    mesh = plsc.VectorSubcoreMesh(core_axis_name="c", subcore_axis_name="s")
    cp = pltpu.CompilerParams(needs_layout_passes=False,
                              use_tc_tiling_on_sc=True)
    kern = functools.partial(
        pl.kernel,
        compiler_params=cp,
        out_type=jax.ShapeDtypeStruct((B,), jnp.float32),
        mesh=mesh,
        scratch_types=[
            pltpu.VMEM((b_per_w,), jnp.int32),
            pltpu.VMEM((b_per_w,), jnp.int32),
            pltpu.VMEM((b_per_w,), jnp.int32),
            pltpu.VMEM((NRING, L, F, 128), jnp.float32),
            pltpu.VMEM((2, WCHUNK, 8 * F), jnp.float32),
            pltpu.VMEM((L * F,), jnp.float32),
            pltpu.VMEM((b_per_w * F,), jnp.float32),
            pltpu.VMEM((b_per_w,), jnp.float32),
            pltpu.SemaphoreType.DMA((NRING + 2,)),
        ],
    )(_mf_kernel_body)
    return kern(task, worker, tfT, wfp)


def kernel(task, worker, task_factors, worker_factors):
    return _mf_forward(task, worker, task_factors, worker_factors)
